# baseline (speedup 1.0000x reference)
import functools

import jax
import jax.numpy as jnp
import numpy as np
from jax import lax
from jax.experimental import pallas as pl
from jax.experimental.pallas import tpu as pltpu
from jax.experimental.pallas import tpu_sc as plsc

FEAT = 100000
D = 64
B = 4096
F = 26
NTOT = B * F
NW = 32
PER_W = NTOT // NW
CH = 128
NCH = PER_W // CH
NPAD = 100352


def _mask_body(mwi_ref, mws_ref, mwj_ref, m1_ref, m2_ref):
    si = jax.nn.sigmoid(mwi_ref[...])
    ss = jax.nn.sigmoid(mws_ref[...])
    sj = jax.nn.sigmoid(mwj_ref[...])
    use_alt = ss < 0.5
    m1_ref[...] = jnp.where(use_alt, si, ss)
    m2_ref[...] = jnp.where(use_alt, sj, ss)


def _make_masks(mwi, mws, mwj):
    shp = jax.ShapeDtypeStruct((NPAD // 128, 128), jnp.float32)
    return pl.pallas_call(
        _mask_body,
        out_shape=(shp, shp),
    )(mwi, mws, mwj)


def _sc_body(idx_hbm, emb_hbm, m1_hbm, m2_hbm, out1_hbm, out2_hbm,
             idx_v, m1_v, m2_v, emb_v, o1_v, o2_v, sem):
    c = lax.axis_index("c")
    s = lax.axis_index("s")
    wid = s * 2 + c
    base = wid * PER_W

    def chunk_body(ci, carry):
        off = base + ci * CH
        pltpu.sync_copy(idx_hbm.at[pl.ds(off, CH)], idx_v)
        cp_e = pltpu.async_copy(emb_hbm.at[idx_v], emb_v, sem)
        cp_1 = pltpu.async_copy(m1_hbm.at[idx_v], m1_v, sem)
        cp_2 = pltpu.async_copy(m2_hbm.at[idx_v], m2_v, sem)
        cp_e.wait()
        cp_1.wait()
        cp_2.wait()

        def row_body(r, carry2):
            ridx = jnp.full((16,), r, jnp.int32)
            m1 = plsc.load_gather(m1_v, [ridx])
            m2 = plsc.load_gather(m2_v, [ridx])
            for k in range(4):
                e = emb_v[r, pl.ds(k * 16, 16)]
                o1_v[r, pl.ds(k * 16, 16)] = e * m1
                o2_v[r, pl.ds(k * 16, 16)] = e * m2
            return carry2

        lax.fori_loop(0, CH, row_body, 0)
        pltpu.sync_copy(o1_v, out1_hbm.at[pl.ds(off, CH)])
        pltpu.sync_copy(o2_v, out2_hbm.at[pl.ds(off, CH)])
        return carry

    lax.fori_loop(0, NCH, chunk_body, 0)


@functools.partial(jax.jit, static_argnums=())
def kernel(x, embedding, mask_weight_i, mask_weight_s, mask_weight_j):
    x_flat = x.reshape(-1).astype(jnp.int32)

    def pad128(w):
        v = w.reshape(-1)
        return jnp.pad(v, (0, NPAD - FEAT)).reshape(NPAD // 128, 128)

    m1, m2 = _make_masks(pad128(mask_weight_i), pad128(mask_weight_s),
                         pad128(mask_weight_j))
    m1f = m1.reshape(-1)[:FEAT]
    m2f = m2.reshape(-1)[:FEAT]

    sc = pl.kernel(
        _sc_body,
        mesh=plsc.VectorSubcoreMesh(core_axis_name="c", subcore_axis_name="s"),
        compiler_params=pltpu.CompilerParams(needs_layout_passes=False,
                                             use_tc_tiling_on_sc=False),
        out_type=[jax.ShapeDtypeStruct((NTOT, D), jnp.float32),
                  jax.ShapeDtypeStruct((NTOT, D), jnp.float32)],
        scratch_types=[
            pltpu.VMEM((CH,), jnp.int32),
            pltpu.VMEM((CH,), jnp.float32),
            pltpu.VMEM((CH,), jnp.float32),
            pltpu.VMEM((CH, D), jnp.float32),
            pltpu.VMEM((CH, D), jnp.float32),
            pltpu.VMEM((CH, D), jnp.float32),
            pltpu.SemaphoreType.DMA,
        ],
    )
    out1, out2 = sc(x_flat, embedding, m1f, m2f)
    return (out1.reshape(B, F, D), out2.reshape(B, F, D))


# speedup vs baseline: 1.2146x; 1.2146x over previous
import functools

import jax
import jax.numpy as jnp
import numpy as np
from jax import lax
from jax.experimental import pallas as pl
from jax.experimental.pallas import tpu as pltpu
from jax.experimental.pallas import tpu_sc as plsc

FEAT = 100000
D = 64
B = 4096
F = 26
NTOT = B * F
NW = 32
PER_W = NTOT // NW
CH = 128
NCH = PER_W // CH
NPAD = 100352


def _mask_body(mwi_ref, mws_ref, mwj_ref, m1_ref, m2_ref):
    si = jax.nn.sigmoid(mwi_ref[...])
    ss = jax.nn.sigmoid(mws_ref[...])
    sj = jax.nn.sigmoid(mwj_ref[...])
    use_alt = ss < 0.5
    m1_ref[...] = jnp.where(use_alt, si, ss)
    m2_ref[...] = jnp.where(use_alt, sj, ss)


def _make_masks(mwi, mws, mwj):
    shp = jax.ShapeDtypeStruct((NPAD // 128, 128), jnp.float32)
    return pl.pallas_call(
        _mask_body,
        out_shape=(shp, shp),
    )(mwi, mws, mwj)


def _sc_body(idx_hbm, emb_hbm, m1_hbm, m2_hbm, out1_hbm, out2_hbm,
             all_idx_v,
             emb_v0, emb_v1, m1_v0, m1_v1, m2_v0, m2_v1,
             o1_v0, o1_v1, o2_v0, o2_v1,
             sem_in0, sem_in1, sem_out0, sem_out1):
    c = lax.axis_index("c")
    s = lax.axis_index("s")
    wid = s * 2 + c
    base = wid * PER_W

    emb_v = (emb_v0, emb_v1)
    m1_v = (m1_v0, m1_v1)
    m2_v = (m2_v0, m2_v1)
    o1_v = (o1_v0, o1_v1)
    o2_v = (o2_v0, o2_v1)
    sem_in = (sem_in0, sem_in1)
    sem_out = (sem_out0, sem_out1)

    def in_copies(ci, b):
        idxr = all_idx_v.at[ci]
        return (
            pltpu.make_async_copy(emb_hbm.at[idxr], emb_v[b], sem_in[b]),
            pltpu.make_async_copy(m1_hbm.at[idxr], m1_v[b], sem_in[b]),
            pltpu.make_async_copy(m2_hbm.at[idxr], m2_v[b], sem_in[b]),
        )

    def out_copies(ci, b):
        off = base + ci * CH
        return (
            pltpu.make_async_copy(o1_v[b], out1_hbm.at[pl.ds(off, CH)],
                                  sem_out[b]),
            pltpu.make_async_copy(o2_v[b], out2_hbm.at[pl.ds(off, CH)],
                                  sem_out[b]),
        )

    def fire(copies):
        for cp in copies:
            cp.start()

    def drain(copies):
        for cp in copies:
            cp.wait()

    pltpu.sync_copy(idx_hbm.at[pl.ds(wid * NCH, NCH)], all_idx_v)
    fire(in_copies(0, 0))
    fire(in_copies(1, 1))

    def compute(b):
        def row_body(r, carry2):
            ridx = jnp.full((16,), r, jnp.int32)
            m1 = plsc.load_gather(m1_v[b], [ridx])
            m2 = plsc.load_gather(m2_v[b], [ridx])
            for k in range(4):
                e = emb_v[b][r, pl.ds(k * 16, 16)]
                o1_v[b][r, pl.ds(k * 16, 16)] = e * m1
                o2_v[b][r, pl.ds(k * 16, 16)] = e * m2
            return carry2

        lax.fori_loop(0, CH, row_body, 0)

    def outer(g, carry):
        for b in (0, 1):
            ci = 2 * g + b
            drain(in_copies(ci, b))

            @pl.when(g >= 1)
            def _():
                drain(out_copies(ci - 2, b))

            compute(b)
            fire(out_copies(ci, b))

            @pl.when(g < NCH // 2 - 1)
            def _():
                fire(in_copies(ci + 2, b))

        return carry

    lax.fori_loop(0, NCH // 2, outer, 0)
    drain(out_copies(NCH - 2, 0))
    drain(out_copies(NCH - 1, 1))


@functools.partial(jax.jit, static_argnums=())
def kernel(x, embedding, mask_weight_i, mask_weight_s, mask_weight_j):
    x_flat = x.reshape(-1).astype(jnp.int32)

    def pad128(w):
        v = w.reshape(-1)
        return jnp.pad(v, (0, NPAD - FEAT)).reshape(NPAD // 128, 128)

    m1, m2 = _make_masks(pad128(mask_weight_i), pad128(mask_weight_s),
                         pad128(mask_weight_j))
    m1f = m1.reshape(-1)[:FEAT]
    m2f = m2.reshape(-1)[:FEAT]

    sc = pl.kernel(
        _sc_body,
        mesh=plsc.VectorSubcoreMesh(core_axis_name="c", subcore_axis_name="s"),
        compiler_params=pltpu.CompilerParams(needs_layout_passes=False,
                                             use_tc_tiling_on_sc=False),
        out_type=[jax.ShapeDtypeStruct((NTOT, D), jnp.float32),
                  jax.ShapeDtypeStruct((NTOT, D), jnp.float32)],
        scratch_types=[
            pltpu.VMEM((NCH, CH), jnp.int32),
            pltpu.VMEM((CH, D), jnp.float32),
            pltpu.VMEM((CH, D), jnp.float32),
            pltpu.VMEM((CH,), jnp.float32),
            pltpu.VMEM((CH,), jnp.float32),
            pltpu.VMEM((CH,), jnp.float32),
            pltpu.VMEM((CH,), jnp.float32),
            pltpu.VMEM((CH, D), jnp.float32),
            pltpu.VMEM((CH, D), jnp.float32),
            pltpu.VMEM((CH, D), jnp.float32),
            pltpu.VMEM((CH, D), jnp.float32),
            pltpu.SemaphoreType.DMA,
            pltpu.SemaphoreType.DMA,
            pltpu.SemaphoreType.DMA,
            pltpu.SemaphoreType.DMA,
        ],
    )
    out1, out2 = sc(x_flat.reshape(NW * NCH, CH), embedding, m1f, m2f)
    return (out1.reshape(B, F, D), out2.reshape(B, F, D))


# speedup vs baseline: 1.2162x; 1.0013x over previous
import functools

import jax
import jax.numpy as jnp
import numpy as np
from jax import lax
from jax.experimental import pallas as pl
from jax.experimental.pallas import tpu as pltpu
from jax.experimental.pallas import tpu_sc as plsc

FEAT = 100000
D = 64
B = 4096
F = 26
NTOT = B * F
NW = 32
PER_W = NTOT // NW
CH = 128
NCH = PER_W // CH
NPAD = 100352


def _mask_body(mwi_ref, mws_ref, mwj_ref, m1_ref, m2_ref):
    si = jax.nn.sigmoid(mwi_ref[...])
    ss = jax.nn.sigmoid(mws_ref[...])
    sj = jax.nn.sigmoid(mwj_ref[...])
    use_alt = ss < 0.5
    m1_ref[...] = jnp.where(use_alt, si, ss)
    m2_ref[...] = jnp.where(use_alt, sj, ss)


def _make_masks(mwi, mws, mwj):
    shp = jax.ShapeDtypeStruct((NPAD // 128, 128), jnp.float32)
    return pl.pallas_call(
        _mask_body,
        out_shape=(shp, shp),
    )(mwi, mws, mwj)


def _sc_body(idx_hbm, emb_hbm, m1_hbm, m2_hbm, out1_hbm, out2_hbm,
             all_idx_v,
             emb_v0, emb_v1, m1_v0, m1_v1, m2_v0, m2_v1,
             o1_v0, o1_v1, o2_v0, o2_v1,
             sem_in0, sem_in1, sem_out0, sem_out1):
    c = lax.axis_index("c")
    s = lax.axis_index("s")
    wid = s * 2 + c
    base = wid * PER_W

    emb_v = (emb_v0, emb_v1)
    m1_v = (m1_v0, m1_v1)
    m2_v = (m2_v0, m2_v1)
    o1_v = (o1_v0, o1_v1)
    o2_v = (o2_v0, o2_v1)
    sem_in = (sem_in0, sem_in1)
    sem_out = (sem_out0, sem_out1)

    def in_copies(ci, b):
        idxr = all_idx_v.at[ci]
        return (
            pltpu.make_async_copy(emb_hbm.at[idxr], emb_v[b], sem_in[b]),
            pltpu.make_async_copy(m1_hbm.at[idxr], m1_v[b], sem_in[b]),
            pltpu.make_async_copy(m2_hbm.at[idxr], m2_v[b], sem_in[b]),
        )

    def out_copies(ci, b):
        off = (base + ci * CH) * D
        return (
            pltpu.make_async_copy(o1_v[b], out1_hbm.at[pl.ds(off, CH * D)],
                                  sem_out[b]),
            pltpu.make_async_copy(o2_v[b], out2_hbm.at[pl.ds(off, CH * D)],
                                  sem_out[b]),
        )

    def fire(copies):
        for cp in copies:
            cp.start()

    def drain(copies):
        for cp in copies:
            cp.wait()

    pltpu.sync_copy(idx_hbm.at[pl.ds(wid * NCH, NCH)], all_idx_v)
    fire(in_copies(0, 0))
    fire(in_copies(1, 1))

    def compute(b):
        def row_body(r, carry2):
            ridx = jnp.full((16,), r, jnp.int32)
            m1 = plsc.load_gather(m1_v[b], [ridx])
            m2 = plsc.load_gather(m2_v[b], [ridx])
            for k in range(4):
                e = emb_v[b][r, pl.ds(k * 16, 16)]
                o1_v[b][pl.ds(r * D + k * 16, 16)] = e * m1
                o2_v[b][pl.ds(r * D + k * 16, 16)] = e * m2
            return carry2

        lax.fori_loop(0, CH, row_body, 0)

    def outer(g, carry):
        for b in (0, 1):
            ci = 2 * g + b
            drain(in_copies(ci, b))

            @pl.when(g >= 1)
            def _():
                drain(out_copies(ci - 2, b))

            compute(b)
            fire(out_copies(ci, b))

            @pl.when(g < NCH // 2 - 1)
            def _():
                fire(in_copies(ci + 2, b))

        return carry

    lax.fori_loop(0, NCH // 2, outer, 0)
    drain(out_copies(NCH - 2, 0))
    drain(out_copies(NCH - 1, 1))


@functools.partial(jax.jit, static_argnums=())
def kernel(x, embedding, mask_weight_i, mask_weight_s, mask_weight_j):
    x_flat = x.reshape(-1).astype(jnp.int32)

    def pad128(w):
        v = w.reshape(-1)
        return jnp.pad(v, (0, NPAD - FEAT)).reshape(NPAD // 128, 128)

    m1, m2 = _make_masks(pad128(mask_weight_i), pad128(mask_weight_s),
                         pad128(mask_weight_j))
    m1f = m1.reshape(-1)[:FEAT]
    m2f = m2.reshape(-1)[:FEAT]

    sc = pl.kernel(
        _sc_body,
        mesh=plsc.VectorSubcoreMesh(core_axis_name="c", subcore_axis_name="s"),
        compiler_params=pltpu.CompilerParams(needs_layout_passes=False,
                                             use_tc_tiling_on_sc=False),
        out_type=[jax.ShapeDtypeStruct((NTOT * D,), jnp.float32),
                  jax.ShapeDtypeStruct((NTOT * D,), jnp.float32)],
        scratch_types=[
            pltpu.VMEM((NCH, CH), jnp.int32),
            pltpu.VMEM((CH, D), jnp.float32),
            pltpu.VMEM((CH, D), jnp.float32),
            pltpu.VMEM((CH,), jnp.float32),
            pltpu.VMEM((CH,), jnp.float32),
            pltpu.VMEM((CH,), jnp.float32),
            pltpu.VMEM((CH,), jnp.float32),
            pltpu.VMEM((CH * D,), jnp.float32),
            pltpu.VMEM((CH * D,), jnp.float32),
            pltpu.VMEM((CH * D,), jnp.float32),
            pltpu.VMEM((CH * D,), jnp.float32),
            pltpu.SemaphoreType.DMA,
            pltpu.SemaphoreType.DMA,
            pltpu.SemaphoreType.DMA,
            pltpu.SemaphoreType.DMA,
        ],
    )
    out1, out2 = sc(x_flat.reshape(NW * NCH, CH), embedding, m1f, m2f)
    return (out1.reshape(B, F, D), out2.reshape(B, F, D))
